# baseline (speedup 1.0000x reference)
import functools

import jax
import jax.numpy as jnp
from jax import lax
from jax.experimental import pallas as pl
from jax.experimental.pallas import tpu as pltpu
from jax.experimental.pallas import tpu_sc as plsc

VOCAB = 1000000
EMB_DIM = 16
BATCH = 16384
HIST = 200

NC, NS, L = 2, 16, 16
NW = NC * NS
ROWS_PER_W = BATCH // NW
CH = 128
NCHUNK = ROWS_PER_W // CH
CHW = CH * HIST

SBLK = 16384
NSTEP = 62
VOCABP = NSTEP * SBLK


def _score_body(e_ref, w_ref, o_ref):
    w128 = jnp.broadcast_to(w_ref[...], (128, EMB_DIM)).astype(jnp.bfloat16)
    full = jax.lax.dot_general(
        w128, e_ref[...].astype(jnp.bfloat16), (((1,), (0,)), ((), ())),
        preferred_element_type=jnp.float32)
    rows = [full[r:r + 1, 128 * r:128 * (r + 1)] for r in range(128)]
    o_ref[0] = jnp.concatenate(rows, axis=0)


def _compute_scores(emb, W):
    scores3 = pl.pallas_call(
        _score_body,
        grid=(NSTEP,),
        in_specs=[
            pl.BlockSpec((EMB_DIM, SBLK), lambda i: (0, i)),
            pl.BlockSpec((1, EMB_DIM), lambda i: (0, 0)),
        ],
        out_specs=pl.BlockSpec((1, 128, 128), lambda i: (i, 0, 0)),
        out_shape=jax.ShapeDtypeStruct((NSTEP, 128, 128), jnp.float32),
    )(emb.T, W)
    return scores3.reshape(VOCABP)


def _sc_body(xT, scores_hbm, b16, out_hbm, scores_sh, idx2t_v,
             vals2t_v, bounce_v, b_v, outbuf, semA, semB):
    cid = lax.axis_index("c")
    sid = lax.axis_index("s")
    wid = sid * NC + cid

    q = VOCABP // NS
    NP = 8
    sizes = (8000,) * 7 + (q - 7 * 8000,)
    offs = tuple(8000 * i for i in range(NP))
    halves = (0, 8000)
    cin = [None] * NP
    cout = [None] * NP
    cin[0] = pltpu.async_copy(scores_hbm.at[pl.ds(sid * q, sizes[0])],
                              bounce_v.at[pl.ds(0, sizes[0])], semA)
    for i in range(NP):
        if i + 1 < NP:
            if i >= 1:
                cout[i - 1].wait()
            cin[i + 1] = pltpu.async_copy(
                scores_hbm.at[pl.ds(sid * q + offs[i + 1], sizes[i + 1])],
                bounce_v.at[pl.ds(halves[(i + 1) % 2], sizes[i + 1])], semA)
        cin[i].wait()
        cout[i] = pltpu.async_copy(
            bounce_v.at[pl.ds(halves[i % 2], sizes[i])],
            scores_sh.at[pl.ds(sid * q + offs[i], sizes[i])], semB)
    cout[NP - 2].wait()
    cout[NP - 1].wait()

    pltpu.sync_copy(b16, b_v)
    plsc.subcore_barrier()

    bvec = b_v[...]
    inv_hist = jnp.float32(1.0 / HIST)
    zero = jnp.zeros((L,), jnp.float32)

    for ci in range(NCHUNK):
        cbase = wid * ROWS_PER_W + ci * CH
        pltpu.sync_copy(xT.at[:, pl.ds(cbase, CH)], idx2t_v)
        copies = [
            pltpu.async_copy(scores_sh.at[idx2t_v.at[l]],
                             vals2t_v.at[l], semA)
            for l in range(HIST)
        ]
        for c in copies:
            c.wait()

        for g in range(CH // L):

            def red(j, accs, g=g):
                a0, a1, a2, a3 = accs
                a0 = a0 + vals2t_v[4 * j, pl.ds(g * L, L)]
                a1 = a1 + vals2t_v[4 * j + 1, pl.ds(g * L, L)]
                a2 = a2 + vals2t_v[4 * j + 2, pl.ds(g * L, L)]
                a3 = a3 + vals2t_v[4 * j + 3, pl.ds(g * L, L)]
                return (a0, a1, a2, a3)

            z4 = lax.fori_loop(0, HIST // 4, red, (zero, zero, zero, zero))
            s = (z4[0] + z4[1]) + (z4[2] + z4[3])
            z = s * inv_hist + bvec
            y = 1.0 / (1.0 + jnp.exp(-z))
            r = y * 10000.0
            t = r.astype(jnp.int32)
            tf = t.astype(jnp.float32)
            frac = r - tf
            odd = (t & 1) == 1
            up = (frac > 0.5) | ((frac == 0.5) & odd)
            outbuf[pl.ds(g * L, L)] = jnp.where(up, tf + 1.0, tf) / 10000.0

        pltpu.sync_copy(outbuf, out_hbm.at[pl.ds(cbase, CH)])


_sc_kernel = functools.partial(
    pl.kernel,
    out_type=jax.ShapeDtypeStruct((BATCH,), jnp.float32),
    mesh=plsc.VectorSubcoreMesh(core_axis_name="c", subcore_axis_name="s",
                                num_cores=NC, num_subcores=NS),
    scratch_types=[
        pltpu.VMEM_SHARED((VOCABP,), jnp.float32),
        pltpu.VMEM((HIST, CH), jnp.int32),
        pltpu.VMEM((HIST, CH), jnp.float32),
        pltpu.VMEM((16000,), jnp.float32),
        pltpu.VMEM((L,), jnp.float32),
        pltpu.VMEM((CH,), jnp.float32),
        pltpu.SemaphoreType.DMA,
        pltpu.SemaphoreType.DMA,
    ],
    compiler_params=pltpu.CompilerParams(needs_layout_passes=False),
)(_sc_body)


def kernel(x, emb, W, b):
    scores = _compute_scores(emb, W)
    b16 = jnp.broadcast_to(b.reshape(1).astype(jnp.float32), (L,))
    out = _sc_kernel(x.T, scores, b16)
    return out.reshape(BATCH, 1)


# speedup vs baseline: 1.2568x; 1.2568x over previous
import functools

import jax
import jax.numpy as jnp
from jax import lax
from jax.experimental import pallas as pl
from jax.experimental.pallas import tpu as pltpu
from jax.experimental.pallas import tpu_sc as plsc

VOCAB = 1000000
EMB_DIM = 16
BATCH = 16384
HIST = 200

NC, NS, L = 2, 16, 16
NW = NC * NS
ROWS_PER_W = BATCH // NW
CH = 128
NCHUNK = ROWS_PER_W // CH
CHW = CH * HIST

SBLK = 16384
NSTEP = 62
VOCABP = NSTEP * SBLK


def _score_body(e_ref, w_ref, o_ref):
    e = e_ref[...]
    w = w_ref[...]
    rows = [
        jnp.sum(e[:, 128 * r:128 * (r + 1)] * w, axis=0, keepdims=True)
        for r in range(128)
    ]
    o_ref[0] = jnp.concatenate(rows, axis=0)


def _compute_scores(emb, W):
    scores3 = pl.pallas_call(
        _score_body,
        grid=(NSTEP,),
        in_specs=[
            pl.BlockSpec((EMB_DIM, SBLK), lambda i: (0, i)),
            pl.BlockSpec((EMB_DIM, 128), lambda i: (0, 0)),
        ],
        out_specs=pl.BlockSpec((1, 128, 128), lambda i: (i, 0, 0)),
        out_shape=jax.ShapeDtypeStruct((NSTEP, 128, 128), jnp.float32),
    )(emb.T, jnp.broadcast_to(W.reshape(EMB_DIM, 1), (EMB_DIM, 128)))
    return scores3.reshape(VOCABP)


def _sc_body(xT, scores_hbm, b16, out_hbm, scores_sh, idx2t_v,
             vals2t_v, bounce_v, b_v, outbuf, semA, semB):
    cid = lax.axis_index("c")
    sid = lax.axis_index("s")
    wid = sid * NC + cid

    q = VOCABP // NS
    NP = 8
    sizes = (8000,) * 7 + (q - 7 * 8000,)
    offs = tuple(8000 * i for i in range(NP))
    halves = (0, 8000)
    cin = [None] * NP
    cout = [None] * NP
    cin[0] = pltpu.async_copy(scores_hbm.at[pl.ds(sid * q, sizes[0])],
                              bounce_v.at[pl.ds(0, sizes[0])], semA)
    for i in range(NP):
        if i + 1 < NP:
            if i >= 1:
                cout[i - 1].wait()
            cin[i + 1] = pltpu.async_copy(
                scores_hbm.at[pl.ds(sid * q + offs[i + 1], sizes[i + 1])],
                bounce_v.at[pl.ds(halves[(i + 1) % 2], sizes[i + 1])], semA)
        cin[i].wait()
        cout[i] = pltpu.async_copy(
            bounce_v.at[pl.ds(halves[i % 2], sizes[i])],
            scores_sh.at[pl.ds(sid * q + offs[i], sizes[i])], semB)
    cout[NP - 2].wait()
    cout[NP - 1].wait()

    pltpu.sync_copy(b16, b_v)
    plsc.subcore_barrier()

    bvec = b_v[...]
    inv_hist = jnp.float32(1.0 / HIST)
    zero = jnp.zeros((L,), jnp.float32)

    for ci in range(NCHUNK):
        cbase = wid * ROWS_PER_W + ci * CH
        pltpu.sync_copy(xT.at[:, pl.ds(cbase, CH)], idx2t_v)
        copies = [
            pltpu.async_copy(scores_sh.at[idx2t_v.at[l]],
                             vals2t_v.at[l], semA)
            for l in range(HIST)
        ]
        for c in copies:
            c.wait()

        for g in range(CH // L):

            def red(j, accs, g=g):
                a0, a1, a2, a3 = accs
                a0 = a0 + vals2t_v[4 * j, pl.ds(g * L, L)]
                a1 = a1 + vals2t_v[4 * j + 1, pl.ds(g * L, L)]
                a2 = a2 + vals2t_v[4 * j + 2, pl.ds(g * L, L)]
                a3 = a3 + vals2t_v[4 * j + 3, pl.ds(g * L, L)]
                return (a0, a1, a2, a3)

            z4 = lax.fori_loop(0, HIST // 4, red, (zero, zero, zero, zero))
            s = (z4[0] + z4[1]) + (z4[2] + z4[3])
            z = s * inv_hist + bvec
            y = 1.0 / (1.0 + jnp.exp(-z))
            r = y * 10000.0
            t = r.astype(jnp.int32)
            tf = t.astype(jnp.float32)
            frac = r - tf
            odd = (t & 1) == 1
            up = (frac > 0.5) | ((frac == 0.5) & odd)
            outbuf[pl.ds(g * L, L)] = jnp.where(up, tf + 1.0, tf) / 10000.0

        pltpu.sync_copy(outbuf, out_hbm.at[pl.ds(cbase, CH)])


_sc_kernel = functools.partial(
    pl.kernel,
    out_type=jax.ShapeDtypeStruct((BATCH,), jnp.float32),
    mesh=plsc.VectorSubcoreMesh(core_axis_name="c", subcore_axis_name="s",
                                num_cores=NC, num_subcores=NS),
    scratch_types=[
        pltpu.VMEM_SHARED((VOCABP,), jnp.float32),
        pltpu.VMEM((HIST, CH), jnp.int32),
        pltpu.VMEM((HIST, CH), jnp.float32),
        pltpu.VMEM((16000,), jnp.float32),
        pltpu.VMEM((L,), jnp.float32),
        pltpu.VMEM((CH,), jnp.float32),
        pltpu.SemaphoreType.DMA,
        pltpu.SemaphoreType.DMA,
    ],
    compiler_params=pltpu.CompilerParams(needs_layout_passes=False),
)(_sc_body)


def kernel(x, emb, W, b):
    scores = _compute_scores(emb, W)
    b16 = jnp.broadcast_to(b.reshape(1).astype(jnp.float32), (L,))
    out = _sc_kernel(x.T, scores, b16)
    return out.reshape(BATCH, 1)
